# Initial kernel scaffold; baseline (speedup 1.0000x reference)
import jax
import jax.numpy as jnp
from jax.experimental import pallas as pl


def kernel(node_embeddings, edge_index):
    raise NotImplementedError("write your pallas kernel here")


# speedup vs baseline: 3.0508x; 3.0508x over previous
import functools

import jax
import jax.numpy as jnp
from jax import lax
from jax.experimental import pallas as pl
from jax.experimental.pallas import tpu as pltpu
from jax.experimental.pallas import tpu_sc as plsc

NC = 2
NS = 16
L = 16
NW = NC * NS

B = 320000
D = 128
EPW = B // NW
CHUNK = 80
NCHUNK = EPW // CHUNK


def _edge_dot_kernel(table, src_idx, tgt_idx, out,
                     src_idx_v, tgt_idx_v, src_rows, tgt_rows, scores, sem):
    wid = lax.axis_index("s") * NC + lax.axis_index("c")
    base = wid * EPW
    pltpu.sync_copy(src_idx.at[pl.ds(base, EPW)], src_idx_v)
    pltpu.sync_copy(tgt_idx.at[pl.ds(base, EPW)], tgt_idx_v)

    def chunk_body(c, carry):
        off = c * CHUNK
        cp_s = pltpu.async_copy(
            table.at[src_idx_v.at[pl.ds(off, CHUNK)]], src_rows, sem)
        cp_t = pltpu.async_copy(
            table.at[tgt_idx_v.at[pl.ds(off, CHUNK)]], tgt_rows, sem)
        cp_s.wait()
        cp_t.wait()

        lane = lax.iota(jnp.int32, L)

        def group_body(g, carry2):
            group = jnp.zeros((L,), jnp.float32)
            for j in range(L):
                e = g * L + j
                acc = src_rows[e, pl.ds(0, L)] * tgt_rows[e, pl.ds(0, L)]
                for k in range(1, D // L):
                    acc = acc + (src_rows[e, pl.ds(k * L, L)]
                                 * tgt_rows[e, pl.ds(k * L, L)])
                group = jnp.where(lane == j, jnp.sum(acc), group)
            scores[pl.ds(g * L, L)] = group
            return carry2

        lax.fori_loop(0, CHUNK // L, group_body, 0)
        pltpu.sync_copy(scores, out.at[pl.ds(base + off, CHUNK)])
        return carry

    lax.fori_loop(0, NCHUNK, chunk_body, 0)


@functools.partial(
    pl.kernel,
    out_type=jax.ShapeDtypeStruct((B,), jnp.float32),
    mesh=plsc.VectorSubcoreMesh(core_axis_name="c", subcore_axis_name="s"),
    compiler_params=pltpu.CompilerParams(needs_layout_passes=False),
    scratch_types=[
        pltpu.VMEM((EPW,), jnp.int32),
        pltpu.VMEM((EPW,), jnp.int32),
        pltpu.VMEM((CHUNK, D), jnp.float32),
        pltpu.VMEM((CHUNK, D), jnp.float32),
        pltpu.VMEM((CHUNK,), jnp.float32),
        pltpu.SemaphoreType.DMA,
    ],
)
def _edge_dot(table, src_idx, tgt_idx, out, *scratch):
    _edge_dot_kernel(table, src_idx, tgt_idx, out, *scratch)


def kernel(node_embeddings, edge_index):
    src = edge_index[0]
    tgt = edge_index[1]
    return _edge_dot(node_embeddings, src, tgt)


# speedup vs baseline: 4.1138x; 1.3484x over previous
import functools

import jax
import jax.numpy as jnp
from jax import lax
from jax.experimental import pallas as pl
from jax.experimental.pallas import tpu as pltpu
from jax.experimental.pallas import tpu_sc as plsc

NC = 2
NS = 16
L = 16
NW = NC * NS

B = 320000
D = 128
EPW = B // NW
CHUNK = 80
NCHUNK = EPW // CHUNK


def _edge_dot_kernel(table, src_idx, tgt_idx, out,
                     src_idx_v, tgt_idx_v, src_rows, tgt_rows, scores_v,
                     sem0, sem1):
    wid = lax.axis_index("s") * NC + lax.axis_index("c")
    base = wid * EPW
    pltpu.sync_copy(src_idx.at[pl.ds(base, EPW)], src_idx_v)
    pltpu.sync_copy(tgt_idx.at[pl.ds(base, EPW)], tgt_idx_v)

    sems = (sem0, sem1)
    lane = lax.iota(jnp.int32, L)

    def start(c, b):
        off = c * CHUNK
        pltpu.async_copy(
            table.at[src_idx_v.at[pl.ds(off, CHUNK)]], src_rows.at[b],
            sems[b])
        pltpu.async_copy(
            table.at[tgt_idx_v.at[pl.ds(off, CHUNK)]], tgt_rows.at[b],
            sems[b])

    def wait(c, b):
        off = c * CHUNK
        pltpu.make_async_copy(
            table.at[src_idx_v.at[pl.ds(off, CHUNK)]], src_rows.at[b],
            sems[b]).wait()
        pltpu.make_async_copy(
            table.at[tgt_idx_v.at[pl.ds(off, CHUNK)]], tgt_rows.at[b],
            sems[b]).wait()

    def compute(c, b):
        off = c * CHUNK
        sr = src_rows.at[b]
        tr = tgt_rows.at[b]

        def group_body(g, carry2):
            group = jnp.zeros((L,), jnp.float32)
            for j in range(L):
                e = g * L + j
                acc = sr[e, pl.ds(0, L)] * tr[e, pl.ds(0, L)]
                for k in range(1, D // L):
                    acc = acc + (sr[e, pl.ds(k * L, L)]
                                 * tr[e, pl.ds(k * L, L)])
                group = jnp.where(lane == j, jnp.sum(acc), group)
            scores_v[pl.ds(off + g * L, L)] = group
            return carry2

        lax.fori_loop(0, CHUNK // L, group_body, 0)

    start(0, 0)

    def pair_body(i, carry):
        c = 2 * i
        start(c + 1, 1)
        wait(c, 0)
        compute(c, 0)
        start(c + 2, 0)
        wait(c + 1, 1)
        compute(c + 1, 1)
        return carry

    lax.fori_loop(0, NCHUNK // 2, pair_body, 0)
    wait(NCHUNK - 1, 0)
    compute(NCHUNK - 1, 0)

    pltpu.sync_copy(scores_v, out.at[pl.ds(base, EPW)])


@functools.partial(
    pl.kernel,
    out_type=jax.ShapeDtypeStruct((B,), jnp.float32),
    mesh=plsc.VectorSubcoreMesh(core_axis_name="c", subcore_axis_name="s"),
    compiler_params=pltpu.CompilerParams(needs_layout_passes=False),
    scratch_types=[
        pltpu.VMEM((EPW,), jnp.int32),
        pltpu.VMEM((EPW,), jnp.int32),
        pltpu.VMEM((2, CHUNK, D), jnp.float32),
        pltpu.VMEM((2, CHUNK, D), jnp.float32),
        pltpu.VMEM((EPW,), jnp.float32),
        pltpu.SemaphoreType.DMA,
        pltpu.SemaphoreType.DMA,
    ],
)
def _edge_dot(table, src_idx, tgt_idx, out, *scratch):
    _edge_dot_kernel(table, src_idx, tgt_idx, out, *scratch)


def kernel(node_embeddings, edge_index):
    src = edge_index[0]
    tgt = edge_index[1]
    return _edge_dot(node_embeddings, src, tgt)


# speedup vs baseline: 10.0968x; 2.4544x over previous
import functools

import jax
import jax.numpy as jnp
from jax import lax
from jax.experimental import pallas as pl
from jax.experimental.pallas import tpu as pltpu
from jax.experimental.pallas import tpu_sc as plsc

NC = 2
NS = 16
L = 16
NW = NC * NS

B = 320000
D = 128
EPW = B // NW
CHUNK = 80
NCHUNK = EPW // CHUNK


def _edge_dot_kernel(table, src_idx, tgt_idx, out,
                     src_idx_v, tgt_idx_v, src_rows, tgt_rows, scores_v,
                     sem0, sem1):
    wid = lax.axis_index("s") * NC + lax.axis_index("c")
    base = wid * EPW
    pltpu.sync_copy(src_idx.at[pl.ds(base, EPW)], src_idx_v)
    pltpu.sync_copy(tgt_idx.at[pl.ds(base, EPW)], tgt_idx_v)

    sems = (sem0, sem1)
    lane = lax.iota(jnp.int32, L)

    def start(c, b):
        off = c * CHUNK
        pltpu.async_copy(
            table.at[src_idx_v.at[pl.ds(off, CHUNK)]], src_rows.at[b],
            sems[b])
        pltpu.async_copy(
            table.at[tgt_idx_v.at[pl.ds(off, CHUNK)]], tgt_rows.at[b],
            sems[b])

    def wait(c, b):
        off = c * CHUNK
        pltpu.make_async_copy(
            table.at[src_idx_v.at[pl.ds(off, CHUNK)]], src_rows.at[b],
            sems[b]).wait()
        pltpu.make_async_copy(
            table.at[tgt_idx_v.at[pl.ds(off, CHUNK)]], tgt_rows.at[b],
            sems[b]).wait()

    def compute(c, b):
        off = c * CHUNK
        sr = src_rows.at[b]
        tr = tgt_rows.at[b]

        def group_body(g, carry2):
            group = jnp.zeros((L,), jnp.float32)
            for j in range(L):
                e = g * L + j
                acc = None
                for k in range(D // (2 * L)):
                    s = plsc.bitcast(sr[e, pl.ds(k * L, L)], jnp.bfloat16)
                    t = plsc.bitcast(tr[e, pl.ds(k * L, L)], jnp.bfloat16)
                    p0, p1 = plsc.unpack(
                        s * t, format=plsc.PackFormat.INTERLEAVED)
                    ps = p0 + p1
                    acc = ps if acc is None else acc + ps
                group = jnp.where(lane == j, jnp.sum(acc), group)
            scores_v[pl.ds(off + g * L, L)] = group
            return carry2

        lax.fori_loop(0, CHUNK // L, group_body, 0)

    start(0, 0)

    def pair_body(i, carry):
        c = 2 * i
        start(c + 1, 1)
        wait(c, 0)
        compute(c, 0)
        start(c + 2, 0)
        wait(c + 1, 1)
        compute(c + 1, 1)
        return carry

    lax.fori_loop(0, NCHUNK // 2, pair_body, 0)
    wait(NCHUNK - 1, 0)
    compute(NCHUNK - 1, 0)

    pltpu.sync_copy(scores_v, out.at[pl.ds(base, EPW)])


@functools.partial(
    pl.kernel,
    out_type=jax.ShapeDtypeStruct((B,), jnp.float32),
    mesh=plsc.VectorSubcoreMesh(core_axis_name="c", subcore_axis_name="s"),
    compiler_params=pltpu.CompilerParams(
        needs_layout_passes=False, use_tc_tiling_on_sc=False),
    scratch_types=[
        pltpu.VMEM((EPW,), jnp.int32),
        pltpu.VMEM((EPW,), jnp.int32),
        pltpu.VMEM((2, CHUNK, D // 2), jnp.int32),
        pltpu.VMEM((2, CHUNK, D // 2), jnp.int32),
        pltpu.VMEM((EPW,), jnp.float32),
        pltpu.SemaphoreType.DMA,
        pltpu.SemaphoreType.DMA,
    ],
)
def _edge_dot(table, src_idx, tgt_idx, out, *scratch):
    _edge_dot_kernel(table, src_idx, tgt_idx, out, *scratch)


def kernel(node_embeddings, edge_index):
    table_bf16 = node_embeddings.astype(jnp.bfloat16)
    table_i32 = jax.lax.bitcast_convert_type(
        table_bf16.reshape(table_bf16.shape[0], D // 2, 2), jnp.int32)
    src = edge_index[0]
    tgt = edge_index[1]
    return _edge_dot(table_i32, src, tgt)


# speedup vs baseline: 12.5462x; 1.2426x over previous
import functools

import jax
import jax.numpy as jnp
from jax import lax
from jax.experimental import pallas as pl
from jax.experimental.pallas import tpu as pltpu
from jax.experimental.pallas import tpu_sc as plsc

NC = 2
NS = 16
L = 16
NW = NC * NS

B = 320000
D = 128
EPW = B // NW
CHUNK = 80
NCHUNK = EPW // CHUNK


def _edge_dot_kernel(table, src_idx, tgt_idx, out,
                     src_idx_v, tgt_idx_v, src_rows, tgt_rows, scores_v,
                     table_sh, sem0, sem1):
    sid = lax.axis_index("s")
    wid = sid * NC + lax.axis_index("c")
    base = wid * EPW

    rows_per_tile = table_sh.shape[0] // NS
    pltpu.sync_copy(table.at[pl.ds(sid * rows_per_tile, rows_per_tile)],
                    table_sh.at[pl.ds(sid * rows_per_tile, rows_per_tile)])
    pltpu.sync_copy(src_idx.at[pl.ds(base, EPW)], src_idx_v)
    pltpu.sync_copy(tgt_idx.at[pl.ds(base, EPW)], tgt_idx_v)
    plsc.subcore_barrier()

    sems = (sem0, sem1)
    lane = lax.iota(jnp.int32, L)

    def start(c, b):
        off = c * CHUNK
        pltpu.async_copy(
            table_sh.at[src_idx_v.at[pl.ds(off, CHUNK)]], src_rows.at[b],
            sems[b])
        pltpu.async_copy(
            table_sh.at[tgt_idx_v.at[pl.ds(off, CHUNK)]], tgt_rows.at[b],
            sems[b])

    def wait(c, b):
        off = c * CHUNK
        pltpu.make_async_copy(
            table_sh.at[src_idx_v.at[pl.ds(off, CHUNK)]], src_rows.at[b],
            sems[b]).wait()
        pltpu.make_async_copy(
            table_sh.at[tgt_idx_v.at[pl.ds(off, CHUNK)]], tgt_rows.at[b],
            sems[b]).wait()

    def compute(c, b):
        off = c * CHUNK
        sr = src_rows.at[b]
        tr = tgt_rows.at[b]

        def group_body(g, carry2):
            group = jnp.zeros((L,), jnp.float32)
            for j in range(L):
                e = g * L + j
                acc = None
                for k in range(D // (2 * L)):
                    s = plsc.bitcast(sr[e, pl.ds(k * L, L)], jnp.bfloat16)
                    t = plsc.bitcast(tr[e, pl.ds(k * L, L)], jnp.bfloat16)
                    p0, p1 = plsc.unpack(
                        s * t, format=plsc.PackFormat.INTERLEAVED)
                    ps = p0 + p1
                    acc = ps if acc is None else acc + ps
                group = jnp.where(lane == j, jnp.sum(acc), group)
            scores_v[pl.ds(off + g * L, L)] = group
            return carry2

        lax.fori_loop(0, CHUNK // L, group_body, 0)

    start(0, 0)

    def pair_body(i, carry):
        c = 2 * i
        start(c + 1, 1)
        wait(c, 0)
        compute(c, 0)
        start(c + 2, 0)
        wait(c + 1, 1)
        compute(c + 1, 1)
        return carry

    lax.fori_loop(0, NCHUNK // 2, pair_body, 0)
    wait(NCHUNK - 1, 0)
    compute(NCHUNK - 1, 0)

    pltpu.sync_copy(scores_v, out.at[pl.ds(base, EPW)])


@functools.partial(
    pl.kernel,
    out_type=jax.ShapeDtypeStruct((B,), jnp.float32),
    mesh=plsc.VectorSubcoreMesh(core_axis_name="c", subcore_axis_name="s"),
    compiler_params=pltpu.CompilerParams(
        needs_layout_passes=False, use_tc_tiling_on_sc=False),
    scratch_types=[
        pltpu.VMEM((EPW,), jnp.int32),
        pltpu.VMEM((EPW,), jnp.int32),
        pltpu.VMEM((2, CHUNK, D // 2), jnp.int32),
        pltpu.VMEM((2, CHUNK, D // 2), jnp.int32),
        pltpu.VMEM((EPW,), jnp.float32),
        pltpu.VMEM_SHARED((10000, D // 2), jnp.int32),
        pltpu.SemaphoreType.DMA,
        pltpu.SemaphoreType.DMA,
    ],
)
def _edge_dot(table, src_idx, tgt_idx, out, *scratch):
    _edge_dot_kernel(table, src_idx, tgt_idx, out, *scratch)


def kernel(node_embeddings, edge_index):
    table_bf16 = node_embeddings.astype(jnp.bfloat16)
    table_i32 = jax.lax.bitcast_convert_type(
        table_bf16.reshape(table_bf16.shape[0], D // 2, 2), jnp.int32)
    src = edge_index[0]
    tgt = edge_index[1]
    return _edge_dot(table_i32, src, tgt)


# speedup vs baseline: 17.1538x; 1.3673x over previous
import functools

import jax
import jax.numpy as jnp
from jax import lax
from jax.experimental import pallas as pl
from jax.experimental.pallas import tpu as pltpu
from jax.experimental.pallas import tpu_sc as plsc

NC = 2
NS = 16
L = 16
NW = NC * NS

B = 320000
D = 128
EPW = B // NW
CHUNK = 80
NCHUNK = EPW // CHUNK


def _edge_dot_kernel(table, edge_idx, out,
                     src_idx_v, tgt_idx_v, src_rows, tgt_rows, scores_v,
                     table_sh, sem0, sem1):
    sid = lax.axis_index("s")
    wid = sid * NC + lax.axis_index("c")
    base = wid * EPW

    rows_per_tile = table_sh.shape[0] // NS
    pltpu.sync_copy(table.at[pl.ds(sid * rows_per_tile, rows_per_tile)],
                    table_sh.at[pl.ds(sid * rows_per_tile, rows_per_tile)])
    pltpu.sync_copy(edge_idx.at[0, pl.ds(base, EPW)], src_idx_v)
    pltpu.sync_copy(edge_idx.at[1, pl.ds(base, EPW)], tgt_idx_v)
    plsc.subcore_barrier()

    sems = (sem0, sem1)
    lane = lax.iota(jnp.int32, L)

    def start(c, b):
        off = c * CHUNK
        pltpu.async_copy(
            table_sh.at[src_idx_v.at[pl.ds(off, CHUNK)]], src_rows.at[b],
            sems[b])
        pltpu.async_copy(
            table_sh.at[tgt_idx_v.at[pl.ds(off, CHUNK)]], tgt_rows.at[b],
            sems[b])

    def wait(c, b):
        off = c * CHUNK
        pltpu.make_async_copy(
            table_sh.at[src_idx_v.at[pl.ds(off, CHUNK)]], src_rows.at[b],
            sems[b]).wait()
        pltpu.make_async_copy(
            table_sh.at[tgt_idx_v.at[pl.ds(off, CHUNK)]], tgt_rows.at[b],
            sems[b]).wait()

    def compute(c, b):
        off = c * CHUNK
        sr = src_rows.at[b]
        tr = tgt_rows.at[b]

        def group_body(g, carry2):
            group = jnp.zeros((L,), jnp.float32)
            for j in range(L):
                e = g * L + j
                acc = None
                for k in range(D // (2 * L)):
                    s = plsc.bitcast(sr[e, pl.ds(k * L, L)], jnp.bfloat16)
                    t = plsc.bitcast(tr[e, pl.ds(k * L, L)], jnp.bfloat16)
                    p0, p1 = plsc.unpack(
                        s * t, format=plsc.PackFormat.INTERLEAVED)
                    ps = p0 + p1
                    acc = ps if acc is None else acc + ps
                group = jnp.where(lane == j, jnp.sum(acc), group)
            scores_v[pl.ds(off + g * L, L)] = group
            return carry2

        lax.fori_loop(0, CHUNK // L, group_body, 0)

    start(0, 0)

    def pair_body(i, carry):
        c = 2 * i
        start(c + 1, 1)
        wait(c, 0)
        compute(c, 0)
        start(c + 2, 0)
        wait(c + 1, 1)
        compute(c + 1, 1)
        return carry

    lax.fori_loop(0, NCHUNK // 2, pair_body, 0)
    wait(NCHUNK - 1, 0)
    compute(NCHUNK - 1, 0)

    pltpu.sync_copy(scores_v, out.at[pl.ds(base, EPW)])


@functools.partial(
    pl.kernel,
    out_type=jax.ShapeDtypeStruct((B,), jnp.float32),
    mesh=plsc.VectorSubcoreMesh(core_axis_name="c", subcore_axis_name="s"),
    compiler_params=pltpu.CompilerParams(
        needs_layout_passes=False, use_tc_tiling_on_sc=False),
    scratch_types=[
        pltpu.VMEM((EPW,), jnp.int32),
        pltpu.VMEM((EPW,), jnp.int32),
        pltpu.VMEM((2, CHUNK, D // 2), jnp.int32),
        pltpu.VMEM((2, CHUNK, D // 2), jnp.int32),
        pltpu.VMEM((EPW,), jnp.float32),
        pltpu.VMEM_SHARED((10000, D // 2), jnp.int32),
        pltpu.SemaphoreType.DMA,
        pltpu.SemaphoreType.DMA,
    ],
)
def _edge_dot(table, edge_idx, out, *scratch):
    _edge_dot_kernel(table, edge_idx, out, *scratch)


def kernel(node_embeddings, edge_index):
    u = jax.lax.bitcast_convert_type(node_embeddings, jnp.uint32)
    b = (u + jnp.uint32(0x7FFF) + ((u >> 16) & jnp.uint32(1))) >> 16
    packed = (b[:, D // 2:] << 16) | b[:, :D // 2]
    table_i32 = jax.lax.bitcast_convert_type(packed, jnp.int32)
    return _edge_dot(table_i32, edge_index)
